# baseline (speedup 1.0000x reference)
import jax
import jax.numpy as jnp
from jax.experimental import pallas as pl
from jax.experimental.pallas import tpu as pltpu

_B, _A, _C = 32, 8400, 80
_EPS = 1e-10
_ALPHA = 0.75


def _loss_kernel(ps_ref, ts_ref, pdt_ref, tbt_ref, a4_ref, lab_ref,
                 fgr_ref, out_ref):
    b = pl.program_id(0)

    ps = ps_ref[0]
    ts = ts_ref[0]
    lab = lab_ref[0]
    fgr = fgr_ref[0]

    cls_iota = jax.lax.broadcasted_iota(jnp.int32, ps.shape, 1)
    oh = cls_iota == lab
    w = jnp.where(oh, ts, _ALPHA * ps * ps)
    logp = jnp.log(ps)
    log1m = jnp.log(1.0 - ps)
    inner = ts * (logp - log1m) + log1m
    m = inner * w
    iota_row = jax.lax.broadcasted_iota(jnp.int32, (1, _A), 1)
    ones_row = (iota_row >= 0).astype(jnp.float32)
    cls_vec = jax.lax.dot_general(ones_row, m, (((1,), (0,)), ((), ())),
                                  preferred_element_type=jnp.float32)
    s_cls = -jnp.sum(cls_vec)

    pdt = pdt_ref[0]
    tbt = tbt_ref[0]
    a4 = a4_ref[...]
    row_iota = jax.lax.broadcasted_iota(jnp.int32, (4, 1), 0)
    sign = jnp.where(row_iota < 2, -1.0, 1.0).astype(jnp.float32)
    pb = a4 + sign * pdt

    b1x1, b1y1, b1x2, b1y2 = pb[0:1], pb[1:2], pb[2:3], pb[3:4]
    b2x1, b2y1, b2x2, b2y2 = tbt[0:1], tbt[1:2], tbt[2:3], tbt[3:4]
    iw = jnp.clip(jnp.minimum(b1x2, b2x2) - jnp.maximum(b1x1, b2x1), 0.0)
    ih = jnp.clip(jnp.minimum(b1y2, b2y2) - jnp.maximum(b1y1, b2y1), 0.0)
    inter = iw * ih
    w1 = b1x2 - b1x1
    h1 = b1y2 - b1y1
    w2 = b2x2 - b2x1
    h2 = b2y2 - b2y1
    union = w1 * h1 + w2 * h2 - inter + _EPS
    iou = inter / union
    cw = jnp.maximum(b1x2, b2x2) - jnp.minimum(b1x1, b2x1)
    ch = jnp.maximum(b1y2, b2y2) - jnp.minimum(b1y1, b2y1)
    c_area = cw * ch + _EPS
    giou = iou - (c_area - union) / c_area
    r = (1.0 - giou) * fgr
    lhs = jnp.concatenate([ones_row, r], axis=0)
    tvec = jax.lax.dot_general(lhs, ts, (((1,), (0,)), ((), ())),
                               preferred_element_type=jnp.float32)
    s_tss = jnp.sum(tvec[0:1])
    s_iou = jnp.sum(tvec[1:2])

    @pl.when(b == 0)
    def _init():
        out_ref[0] = 0.0
        out_ref[1] = 0.0
        out_ref[2] = 0.0

    out_ref[0] += s_cls
    out_ref[1] += s_iou
    out_ref[2] += s_tss


def kernel(pred_scores, pred_distri, anchor_points_s, target_bboxes,
           target_scores, target_labels, fg_mask):
    pdt = pred_distri.transpose(0, 2, 1)
    tbt = target_bboxes.transpose(0, 2, 1)
    apt = anchor_points_s.T
    a4 = jnp.concatenate([apt, apt], axis=0)
    lab = jnp.where(fg_mask, target_labels, _C).astype(jnp.int32)[..., None]
    fgr = fg_mask.astype(jnp.float32)[:, None, :]

    sums = pl.pallas_call(
        _loss_kernel,
        grid=(_B,),
        in_specs=[
            pl.BlockSpec((1, _A, _C), lambda b: (b, 0, 0)),
            pl.BlockSpec((1, _A, _C), lambda b: (b, 0, 0)),
            pl.BlockSpec((1, 4, _A), lambda b: (b, 0, 0)),
            pl.BlockSpec((1, 4, _A), lambda b: (b, 0, 0)),
            pl.BlockSpec((4, _A), lambda b: (0, 0)),
            pl.BlockSpec((1, _A, 1), lambda b: (b, 0, 0)),
            pl.BlockSpec((1, 1, _A), lambda b: (b, 0, 0)),
        ],
        out_specs=pl.BlockSpec(memory_space=pltpu.SMEM),
        out_shape=jax.ShapeDtypeStruct((3,), jnp.float32),
    )(pred_scores, target_scores, pdt, tbt, a4, lab, fgr)

    return (sums[0] + 2.5 * sums[1]) / sums[2]


# speedup vs baseline: 1.1733x; 1.1733x over previous
import jax
import jax.numpy as jnp
from jax.experimental import pallas as pl
from jax.experimental.pallas import tpu as pltpu

_B, _A, _C = 32, 8400, 80
_EPS = 1e-10
_ALPHA = 0.75


def _loss_kernel(ps_ref, ts_ref, aux_ref, a4_ref, labT_ref, out_ref):
    b = pl.program_id(0)

    ps = ps_ref[0]
    ts = ts_ref[0]
    aux = aux_ref[0]
    fgr = aux[8:9]

    e_col = (jax.lax.broadcasted_iota(jnp.int32, (_B, 1), 0)
             == b).astype(jnp.float32)
    lab_col = jax.lax.dot_general(labT_ref[...], e_col,
                                  (((1,), (0,)), ((), ())),
                                  preferred_element_type=jnp.float32)

    iota_c = jax.lax.broadcasted_iota(jnp.int32, (1, _C), 1)
    oh = iota_c.astype(jnp.float32) == lab_col
    w = jnp.where(oh, ts, _ALPHA * ps * ps)
    logp = jnp.log(ps)
    log1m = jnp.log(1.0 - ps)
    inner = ts * (logp - log1m) + log1m
    m = inner * w
    iota_row = jax.lax.broadcasted_iota(jnp.int32, (1, _A), 1)
    ones_row = (iota_row >= 0).astype(jnp.float32)
    cls_vec = jax.lax.dot_general(ones_row, m, (((1,), (0,)), ((), ())),
                                  preferred_element_type=jnp.float32)
    s_cls = -jnp.sum(cls_vec)

    a4 = a4_ref[...]
    row_iota = jax.lax.broadcasted_iota(jnp.int32, (4, 1), 0)
    sign = jnp.where(row_iota < 2, -1.0, 1.0).astype(jnp.float32)
    pb = a4[0:4] + sign * aux[0:4]
    tbt = aux[4:8]

    b1x1, b1y1, b1x2, b1y2 = pb[0:1], pb[1:2], pb[2:3], pb[3:4]
    b2x1, b2y1, b2x2, b2y2 = tbt[0:1], tbt[1:2], tbt[2:3], tbt[3:4]
    iw = jnp.clip(jnp.minimum(b1x2, b2x2) - jnp.maximum(b1x1, b2x1), 0.0)
    ih = jnp.clip(jnp.minimum(b1y2, b2y2) - jnp.maximum(b1y1, b2y1), 0.0)
    inter = iw * ih
    w1 = b1x2 - b1x1
    h1 = b1y2 - b1y1
    w2 = b2x2 - b2x1
    h2 = b2y2 - b2y1
    union = w1 * h1 + w2 * h2 - inter + _EPS
    iou = inter / union
    cw = jnp.maximum(b1x2, b2x2) - jnp.minimum(b1x1, b2x1)
    ch = jnp.maximum(b1y2, b2y2) - jnp.minimum(b1y1, b2y1)
    c_area = cw * ch + _EPS
    giou = iou - (c_area - union) / c_area
    r = (1.0 - giou) * fgr
    lhs = jnp.concatenate([ones_row, r], axis=0)
    tvec = jax.lax.dot_general(lhs, ts, (((1,), (0,)), ((), ())),
                               preferred_element_type=jnp.float32)
    s_tss = jnp.sum(tvec[0:1])
    s_iou = jnp.sum(tvec[1:2])

    @pl.when(b == 0)
    def _init():
        out_ref[0] = 0.0
        out_ref[1] = 0.0
        out_ref[2] = 0.0

    out_ref[0] += s_cls
    out_ref[1] += s_iou
    out_ref[2] += s_tss


def kernel(pred_scores, pred_distri, anchor_points_s, target_bboxes,
           target_scores, target_labels, fg_mask):
    pdt = pred_distri.transpose(0, 2, 1)
    tbt = target_bboxes.transpose(0, 2, 1)
    lab = jnp.where(fg_mask, target_labels, _C).astype(jnp.float32)
    labT = lab.T
    fgf = fg_mask.astype(jnp.float32)
    zeros7 = jnp.zeros((_B, 7, _A), jnp.float32)
    aux = jnp.concatenate([pdt, tbt, fgf[:, None, :], zeros7], axis=1)
    apt = anchor_points_s.T
    a4 = jnp.concatenate([apt, apt, jnp.zeros((4, _A), jnp.float32)], axis=0)

    sums = pl.pallas_call(
        _loss_kernel,
        grid=(_B,),
        in_specs=[
            pl.BlockSpec((1, _A, _C), lambda b: (b, 0, 0)),
            pl.BlockSpec((1, _A, _C), lambda b: (b, 0, 0)),
            pl.BlockSpec((1, 16, _A), lambda b: (b, 0, 0)),
            pl.BlockSpec((8, _A), lambda b: (0, 0)),
            pl.BlockSpec((_A, _B), lambda b: (0, 0)),
        ],
        out_specs=pl.BlockSpec(memory_space=pltpu.SMEM),
        out_shape=jax.ShapeDtypeStruct((3,), jnp.float32),
    )(pred_scores, target_scores, aux, a4, labT)

    return (sums[0] + 2.5 * sums[1]) / sums[2]


# speedup vs baseline: 3.0042x; 2.5604x over previous
import jax
import jax.numpy as jnp
from jax.experimental import pallas as pl
from jax.experimental.pallas import tpu as pltpu

_B, _A, _C = 32, 8400, 80
_EPS = 1e-10
_ALPHA = 0.75


def _loss_kernel(ps_ref, ts_ref, aux_ref, a4_ref, out_ref):
    b = pl.program_id(0)

    ps = ps_ref[0]
    ts = ts_ref[0]
    aux = aux_ref[0]
    lab_row = aux[8:9]
    fgr = aux[9:10]

    cls_iota = jax.lax.broadcasted_iota(jnp.int32, ps.shape, 0)
    oh = cls_iota == lab_row.astype(jnp.int32)
    w = jnp.where(oh, ts, _ALPHA * ps * ps)
    logp = jnp.log(ps)
    log1m = jnp.log(1.0 - ps)
    inner = ts * (logp - log1m) + log1m
    m = inner * w
    iota_c = jax.lax.broadcasted_iota(jnp.int32, (1, _C), 1)
    ones_c = (iota_c >= 0).astype(jnp.float32)
    row_cls = jax.lax.dot_general(ones_c, m, (((1,), (0,)), ((), ())),
                                  preferred_element_type=jnp.float32)

    bw = jnp.sum(ts, axis=0, keepdims=True)

    a4 = a4_ref[...]
    row_iota = jax.lax.broadcasted_iota(jnp.int32, (4, 1), 0)
    sign = jnp.where(row_iota < 2, -1.0, 1.0).astype(jnp.float32)
    pb = a4[0:4] + sign * aux[0:4]
    tbt = aux[4:8]

    b1x1, b1y1, b1x2, b1y2 = pb[0:1], pb[1:2], pb[2:3], pb[3:4]
    b2x1, b2y1, b2x2, b2y2 = tbt[0:1], tbt[1:2], tbt[2:3], tbt[3:4]
    iw = jnp.clip(jnp.minimum(b1x2, b2x2) - jnp.maximum(b1x1, b2x1), 0.0)
    ih = jnp.clip(jnp.minimum(b1y2, b2y2) - jnp.maximum(b1y1, b2y1), 0.0)
    inter = iw * ih
    w1 = b1x2 - b1x1
    h1 = b1y2 - b1y1
    w2 = b2x2 - b2x1
    h2 = b2y2 - b2y1
    union = w1 * h1 + w2 * h2 - inter + _EPS
    iou = inter / union
    cw = jnp.maximum(b1x2, b2x2) - jnp.minimum(b1x1, b2x1)
    ch = jnp.maximum(b1y2, b2y2) - jnp.minimum(b1y1, b2y1)
    c_area = cw * ch + _EPS
    giou = iou - (c_area - union) / c_area
    row_iou = (1.0 - giou) * fgr * bw

    @pl.when(b == 0)
    def _init():
        out_ref[...] = jnp.zeros((8, _A), jnp.float32)

    out_ref[0:1] += row_cls
    out_ref[1:2] += row_iou
    out_ref[2:3] += bw


def kernel(pred_scores, pred_distri, anchor_points_s, target_bboxes,
           target_scores, target_labels, fg_mask):
    psT = pred_scores.transpose(0, 2, 1)
    tsT = target_scores.transpose(0, 2, 1)
    pdt = pred_distri.transpose(0, 2, 1)
    tbt = target_bboxes.transpose(0, 2, 1)
    lab = jnp.where(fg_mask, target_labels, _C).astype(jnp.float32)
    fgf = fg_mask.astype(jnp.float32)
    zeros6 = jnp.zeros((_B, 6, _A), jnp.float32)
    aux = jnp.concatenate(
        [pdt, tbt, lab[:, None, :], fgf[:, None, :], zeros6], axis=1)
    apt = anchor_points_s.T
    a4 = jnp.concatenate([apt, apt, jnp.zeros((4, _A), jnp.float32)], axis=0)

    rows = pl.pallas_call(
        _loss_kernel,
        grid=(_B,),
        in_specs=[
            pl.BlockSpec((1, _C, _A), lambda b: (b, 0, 0)),
            pl.BlockSpec((1, _C, _A), lambda b: (b, 0, 0)),
            pl.BlockSpec((1, 16, _A), lambda b: (b, 0, 0)),
            pl.BlockSpec((8, _A), lambda b: (0, 0)),
        ],
        out_specs=pl.BlockSpec((8, _A), lambda b: (0, 0)),
        out_shape=jax.ShapeDtypeStruct((8, _A), jnp.float32),
    )(psT, tsT, aux, a4)

    s_cls = -jnp.sum(rows[0])
    s_iou = jnp.sum(rows[1])
    s_tss = jnp.sum(rows[2])
    return (s_cls + 2.5 * s_iou) / s_tss


# speedup vs baseline: 4.4658x; 1.4865x over previous
import jax
import jax.numpy as jnp
from jax.experimental import pallas as pl
from jax.experimental.pallas import tpu as pltpu

_B, _A, _C = 32, 8400, 80
_EPS = 1e-10
_ALPHA = 0.75
_NB = 2
_STEPS = _B // _NB


def _loss_kernel(ps_ref, ts_ref, aux_ref, a4_ref, out_ref):
    step = pl.program_id(0)

    ps = ps_ref[...]
    ts = ts_ref[...]
    aux = aux_ref[...]
    a4 = a4_ref[...]

    iota3 = jax.lax.broadcasted_iota(jnp.int32, (_NB, _C, _A), 1)
    lab3 = aux[:, 8:9, :].astype(jnp.int32)
    oh = iota3 == lab3
    w = jnp.where(oh, ts, _ALPHA * ps * ps)
    logp = jnp.log(ps)
    log1m = jnp.log(1.0 - ps)
    inner = ts * (logp - log1m) + log1m
    m = (inner * w).reshape(_NB * _C, _A)
    iota_c = jax.lax.broadcasted_iota(jnp.int32, (1, _NB * _C), 1)
    ones_c = (iota_c >= 0).astype(jnp.float32)
    row_cls = jax.lax.dot_general(ones_c, m, (((1,), (0,)), ((), ())),
                                  preferred_element_type=jnp.float32)

    row_iota = jax.lax.broadcasted_iota(jnp.int32, (4, 1), 0)
    sign = jnp.where(row_iota < 2, -1.0, 1.0).astype(jnp.float32)
    row_iou = None
    row_tss = None
    for i in range(_NB):
        bw = jnp.sum(ts[i], axis=0, keepdims=True)
        pb = a4[0:4] + sign * aux[i, 0:4]
        tbt = aux[i, 4:8]
        fgr = aux[i, 9:10]
        b1x1, b1y1, b1x2, b1y2 = pb[0:1], pb[1:2], pb[2:3], pb[3:4]
        b2x1, b2y1, b2x2, b2y2 = tbt[0:1], tbt[1:2], tbt[2:3], tbt[3:4]
        iw = jnp.clip(jnp.minimum(b1x2, b2x2) - jnp.maximum(b1x1, b2x1), 0.0)
        ih = jnp.clip(jnp.minimum(b1y2, b2y2) - jnp.maximum(b1y1, b2y1), 0.0)
        inter = iw * ih
        w1 = b1x2 - b1x1
        h1 = b1y2 - b1y1
        w2 = b2x2 - b2x1
        h2 = b2y2 - b2y1
        union = w1 * h1 + w2 * h2 - inter + _EPS
        iou = inter / union
        cw = jnp.maximum(b1x2, b2x2) - jnp.minimum(b1x1, b2x1)
        ch = jnp.maximum(b1y2, b2y2) - jnp.minimum(b1y1, b2y1)
        c_area = cw * ch + _EPS
        giou = iou - (c_area - union) / c_area
        contrib = (1.0 - giou) * fgr * bw
        row_iou = contrib if row_iou is None else row_iou + contrib
        row_tss = bw if row_tss is None else row_tss + bw

    @pl.when(step == 0)
    def _init():
        out_ref[...] = jnp.zeros((8, _A), jnp.float32)

    out_ref[0:1] += row_cls
    out_ref[1:2] += row_iou
    out_ref[2:3] += row_tss


def kernel(pred_scores, pred_distri, anchor_points_s, target_bboxes,
           target_scores, target_labels, fg_mask):
    psT = pred_scores.transpose(0, 2, 1)
    tsT = target_scores.transpose(0, 2, 1)
    pdt = pred_distri.transpose(0, 2, 1)
    tbt = target_bboxes.transpose(0, 2, 1)
    lab = jnp.where(fg_mask, target_labels, _C).astype(jnp.float32)
    fgf = fg_mask.astype(jnp.float32)
    aux = jnp.concatenate(
        [pdt, tbt, lab[:, None, :], fgf[:, None, :]], axis=1)
    apt = anchor_points_s.T
    a4 = jnp.concatenate([apt, apt, jnp.zeros((4, _A), jnp.float32)], axis=0)

    rows = pl.pallas_call(
        _loss_kernel,
        grid=(_STEPS,),
        in_specs=[
            pl.BlockSpec((_NB, _C, _A), lambda b: (b, 0, 0)),
            pl.BlockSpec((_NB, _C, _A), lambda b: (b, 0, 0)),
            pl.BlockSpec((_NB, 10, _A), lambda b: (b, 0, 0)),
            pl.BlockSpec((8, _A), lambda b: (0, 0)),
        ],
        out_specs=pl.BlockSpec((8, _A), lambda b: (0, 0)),
        out_shape=jax.ShapeDtypeStruct((8, _A), jnp.float32),
    )(psT, tsT, aux, a4)

    s_cls = -jnp.sum(rows[0])
    s_iou = jnp.sum(rows[1])
    s_tss = jnp.sum(rows[2])
    return (s_cls + 2.5 * s_iou) / s_tss


# speedup vs baseline: 4.7636x; 1.0667x over previous
import jax
import jax.numpy as jnp
from jax.experimental import pallas as pl
from jax.experimental.pallas import tpu as pltpu

_B, _A, _C = 32, 8400, 80
_EPS = 1e-10
_ALPHA = 0.75
_NB = 2
_STEPS = _B // _NB


def _loss_kernel(ps_ref, ts_ref, pd_ref, tb_ref, lab_ref, a4_ref, out_ref):
    step = pl.program_id(0)

    ps = ps_ref[...]
    ts = ts_ref[...]
    a4 = a4_ref[...]

    iota3 = jax.lax.broadcasted_iota(jnp.int32, (_NB, _C, _A), 1)
    lab3 = lab_ref[:, 0:1, :].astype(jnp.int32)
    oh = iota3 == lab3
    w = jnp.where(oh, ts, _ALPHA * ps * ps)
    logp = jnp.log(ps)
    log1m = jnp.log(1.0 - ps)
    inner = ts * (logp - log1m) + log1m
    m = (inner * w).reshape(_NB * _C, _A)
    iota_c = jax.lax.broadcasted_iota(jnp.int32, (1, _NB * _C), 1)
    ones_c = (iota_c >= 0).astype(jnp.float32)
    row_cls = jax.lax.dot_general(ones_c, m, (((1,), (0,)), ((), ())),
                                  preferred_element_type=jnp.float32)

    row_iota = jax.lax.broadcasted_iota(jnp.int32, (4, 1), 0)
    sign = jnp.where(row_iota < 2, -1.0, 1.0).astype(jnp.float32)
    row_iou = None
    row_tss = None
    for i in range(_NB):
        bw = jnp.sum(ts[i], axis=0, keepdims=True)
        pb = a4[0:4] + sign * pd_ref[i]
        tbt = tb_ref[i]
        fgr = (lab_ref[i, 0:1, :] != float(_C)).astype(jnp.float32)
        b1x1, b1y1, b1x2, b1y2 = pb[0:1], pb[1:2], pb[2:3], pb[3:4]
        b2x1, b2y1, b2x2, b2y2 = tbt[0:1], tbt[1:2], tbt[2:3], tbt[3:4]
        iw = jnp.clip(jnp.minimum(b1x2, b2x2) - jnp.maximum(b1x1, b2x1), 0.0)
        ih = jnp.clip(jnp.minimum(b1y2, b2y2) - jnp.maximum(b1y1, b2y1), 0.0)
        inter = iw * ih
        w1 = b1x2 - b1x1
        h1 = b1y2 - b1y1
        w2 = b2x2 - b2x1
        h2 = b2y2 - b2y1
        union = w1 * h1 + w2 * h2 - inter + _EPS
        iou = inter / union
        cw = jnp.maximum(b1x2, b2x2) - jnp.minimum(b1x1, b2x1)
        ch = jnp.maximum(b1y2, b2y2) - jnp.minimum(b1y1, b2y1)
        c_area = cw * ch + _EPS
        giou = iou - (c_area - union) / c_area
        contrib = (1.0 - giou) * fgr * bw
        row_iou = contrib if row_iou is None else row_iou + contrib
        row_tss = bw if row_tss is None else row_tss + bw

    @pl.when(step == 0)
    def _init():
        out_ref[...] = jnp.zeros((8, _A), jnp.float32)

    out_ref[0:1] += row_cls
    out_ref[1:2] += row_iou
    out_ref[2:3] += row_tss


def kernel(pred_scores, pred_distri, anchor_points_s, target_bboxes,
           target_scores, target_labels, fg_mask):
    psT = pred_scores.transpose(0, 2, 1)
    tsT = target_scores.transpose(0, 2, 1)
    pdt = pred_distri.transpose(0, 2, 1)
    tbt = target_bboxes.transpose(0, 2, 1)
    lab = jnp.where(fg_mask, target_labels, _C).astype(jnp.float32)
    lab4 = jnp.concatenate(
        [lab[:, None, :], jnp.zeros((_B, 3, _A), jnp.float32)], axis=1)
    apt = anchor_points_s.T
    a4 = jnp.concatenate([apt, apt, jnp.zeros((4, _A), jnp.float32)], axis=0)

    rows = pl.pallas_call(
        _loss_kernel,
        grid=(_STEPS,),
        in_specs=[
            pl.BlockSpec((_NB, _C, _A), lambda b: (b, 0, 0)),
            pl.BlockSpec((_NB, _C, _A), lambda b: (b, 0, 0)),
            pl.BlockSpec((_NB, 4, _A), lambda b: (b, 0, 0)),
            pl.BlockSpec((_NB, 4, _A), lambda b: (b, 0, 0)),
            pl.BlockSpec((_NB, 4, _A), lambda b: (b, 0, 0)),
            pl.BlockSpec((8, _A), lambda b: (0, 0)),
        ],
        out_specs=pl.BlockSpec((8, _A), lambda b: (0, 0)),
        out_shape=jax.ShapeDtypeStruct((8, _A), jnp.float32),
    )(psT, tsT, pdt, tbt, lab4, a4)

    s_cls = -jnp.sum(rows[0])
    s_iou = jnp.sum(rows[1])
    s_tss = jnp.sum(rows[2])
    return (s_cls + 2.5 * s_iou) / s_tss
